# baseline (speedup 1.0000x reference)
import functools

import jax
import jax.numpy as jnp
from jax import lax
from jax.experimental import pallas as pl
from jax.experimental.pallas import tpu as pltpu
from jax.experimental.pallas import tpu_sc as plsc

N_NODES = 10000
NPAD = 10240
N_EDGES = 160000
TILES = 16
K = 256
CPC = 1
E_PAD = 163840
NCH_A = E_PAD // (TILES * K)
NCH_B = E_PAD // (2 * TILES * K)
NODE_SLICE = NPAD // TILES


def _block_agg(table, srcr, dstr, s, nch, src_v, dst_v, isems,
               rows_v, acc, deg):
    def stage(j, b):
        pltpu.async_copy(srcr.at[s, j], src_v[b], isems[2 * b])
        pltpu.async_copy(dstr.at[s, j], dst_v[b], isems[2 * b + 1])

    def process(j, b, prefetch_j):
        pltpu.make_async_copy(srcr.at[s, j], src_v[b], isems[2 * b]).wait()
        pltpu.make_async_copy(dstr.at[s, j], dst_v[b],
                              isems[2 * b + 1]).wait()
        pltpu.sync_copy(table.at[src_v[b]], rows_v)
        pltpu.sync_copy(rows_v, acc.at[dst_v[b]], add=True)
        if deg is not None:
            ones_v, degacc = deg
            pltpu.sync_copy(ones_v, degacc.at[dst_v[b]], add=True)
        if prefetch_j is not None:
            stage(prefetch_j, b)

    stage(0, 0)
    stage(1, 1)

    @pl.loop(0, nch - 2, step=2)
    def _(g):
        for b in range(2):
            process(g + b, b, g + b + 2)

    for b in range(2):
        process(nch - 2 + b, b, None)


def _sc_agg(table0, table1, src4d, dst4d, zeros2d, zeros1d, with_deg):
    mesh = plsc.VectorSubcoreMesh(core_axis_name="c", subcore_axis_name="s")
    out_type = [jax.ShapeDtypeStruct((NPAD, 128), jnp.float32),
                jax.ShapeDtypeStruct((NPAD, 128), jnp.float32)]
    scratch = ([pltpu.VMEM((K,), jnp.int32) for _ in range(4)]
               + [pltpu.VMEM((K, 128), jnp.float32)]
               + [pltpu.VMEM_SHARED((NPAD, 128), jnp.float32)]
               + [pltpu.SemaphoreType.DMA for _ in range(4)])
    if with_deg:
        out_type.append(jax.ShapeDtypeStruct((NPAD,), jnp.float32))
        scratch.append(pltpu.VMEM((K,), jnp.float32))
        scratch.append(pltpu.VMEM_SHARED((NPAD,), jnp.float32))

    def body(t0, t1, srcr, dstr, z2, z1, *refs):
        if with_deg:
            o0, o1, degout = refs[:3]
            refs = refs[3:]
            ones_v, degacc = refs[-2:]
        else:
            o0, o1 = refs[:2]
            refs = refs[2:]
        src_v = refs[0:2]
        dst_v = refs[2:4]
        rows_v = refs[4]
        acc = refs[5]
        isems = refs[6:10]
        c = lax.axis_index("c")
        s = lax.axis_index("s")
        row0 = s * NODE_SLICE
        pltpu.sync_copy(z2.at[pl.ds(row0, NODE_SLICE)],
                        acc.at[pl.ds(row0, NODE_SLICE)])
        if with_deg:
            @pl.when(c == 0)
            def _():
                pltpu.sync_copy(z1.at[pl.ds(row0, NODE_SLICE)],
                                degacc.at[pl.ds(row0, NODE_SLICE)])
            for j in range(K // 16):
                ones_v[pl.ds(j * 16, 16)] = jnp.full((16,), 1.0, jnp.float32)
        plsc.subcore_barrier()

        deg = (ones_v, degacc) if with_deg else None
        pl.when(c == 0)(lambda: _block_agg(t0, srcr, dstr, s, NCH_A,
                                           src_v, dst_v, isems,
                                           rows_v, acc, deg))
        pl.when(c == 1)(lambda: _block_agg(t1, srcr, dstr, s, NCH_A,
                                           src_v, dst_v, isems,
                                           rows_v, acc, None))
        plsc.subcore_barrier()
        pl.when(c == 0)(lambda: pltpu.sync_copy(
            acc.at[pl.ds(row0, NODE_SLICE)], o0.at[pl.ds(row0, NODE_SLICE)]))
        pl.when(c == 1)(lambda: pltpu.sync_copy(
            acc.at[pl.ds(row0, NODE_SLICE)], o1.at[pl.ds(row0, NODE_SLICE)]))
        if with_deg:
            @pl.when(c == 0)
            def _():
                pltpu.sync_copy(degacc.at[pl.ds(row0, NODE_SLICE)],
                                degout.at[pl.ds(row0, NODE_SLICE)])

    run_kernel = pl.kernel(body, out_type=out_type, mesh=mesh,
                           scratch_types=scratch)
    return run_kernel(table0, table1, src4d, dst4d, zeros2d, zeros1d)


def _sc_agg_edgesplit(table, src4d, dst4d, zeros2d):
    mesh = plsc.VectorSubcoreMesh(core_axis_name="c", subcore_axis_name="s")
    out_type = [jax.ShapeDtypeStruct((NPAD, 128), jnp.float32),
                jax.ShapeDtypeStruct((NPAD, 128), jnp.float32)]
    scratch = ([pltpu.VMEM((K,), jnp.int32) for _ in range(4)]
               + [pltpu.VMEM((K, 128), jnp.float32)]
               + [pltpu.VMEM_SHARED((NPAD, 128), jnp.float32)]
               + [pltpu.SemaphoreType.DMA for _ in range(4)])

    def body(t, srcr, dstr, z2, o0, o1, *refs):
        src_v = refs[0:2]
        dst_v = refs[2:4]
        rows_v = refs[4]
        acc = refs[5]
        isems = refs[6:10]
        c = lax.axis_index("c")
        s = lax.axis_index("s")
        q = c * TILES + s
        row0 = s * NODE_SLICE
        pltpu.sync_copy(z2.at[pl.ds(row0, NODE_SLICE)],
                        acc.at[pl.ds(row0, NODE_SLICE)])
        plsc.subcore_barrier()
        _block_agg(t, srcr, dstr, q, NCH_B, src_v, dst_v, isems,
                   rows_v, acc, None)
        plsc.subcore_barrier()
        pl.when(c == 0)(lambda: pltpu.sync_copy(
            acc.at[pl.ds(row0, NODE_SLICE)], o0.at[pl.ds(row0, NODE_SLICE)]))
        pl.when(c == 1)(lambda: pltpu.sync_copy(
            acc.at[pl.ds(row0, NODE_SLICE)], o1.at[pl.ds(row0, NODE_SLICE)]))

    run_kernel = pl.kernel(body, out_type=out_type, mesh=mesh,
                           scratch_types=scratch)
    return run_kernel(table, src4d, dst4d, zeros2d)


def _dot_t(a, w):
    return lax.dot_general(a, w, (((1,), (1,)), ((), ())),
                           preferred_element_type=jnp.float32)


def _tc_layer_body(with_p, *refs):
    if with_p:
        (a0, a1, t0, t1, deg, wll, wlr, wrl, wrr, b, wp,
         o0, o1, po) = refs
    else:
        a0, a1, t0, t1, deg, wll, wlr, wrl, wrr, b, o0, o1 = refs
    inv = 1.0 / jnp.maximum(deg[...], 1.0)
    h = (_dot_t(a0[...] * inv, wll[...]) + _dot_t(a1[...] * inv, wlr[...])
         + _dot_t(t0[...], wrl[...]) + _dot_t(t1[...], wrr[...]) + b[...])
    h = jnp.maximum(h, 0.0)
    o0[...] = h[:, :128]
    o1[...] = h[:, 128:]
    if with_p:
        po[...] = _dot_t(h, wp[...])


def _tc_layer(a0, a1, t0, t1, deg, Wl, Wr, b, Wp=None):
    BN = 1024
    grid = (NPAD // BN,)
    half = pl.BlockSpec((BN, 128), lambda i: (i, 0))
    wspec = pl.BlockSpec((256, 128), lambda i: (0, 0))
    in_specs = [half, half, half, half,
                pl.BlockSpec((BN, 1), lambda i: (i, 0)),
                wspec, wspec, wspec, wspec,
                pl.BlockSpec((1, 256), lambda i: (0, 0))]
    out_shape = [jax.ShapeDtypeStruct((NPAD, 128), jnp.float32),
                 jax.ShapeDtypeStruct((NPAD, 128), jnp.float32)]
    out_specs = [half, half]
    args = [a0, a1, t0, t1, deg,
            Wl[:, :128], Wl[:, 128:], Wr[:, :128], Wr[:, 128:],
            b.reshape(1, 256)]
    if Wp is not None:
        in_specs.append(pl.BlockSpec((128, 256), lambda i: (0, 0)))
        out_shape.append(jax.ShapeDtypeStruct((NPAD, 128), jnp.float32))
        out_specs.append(half)
        args.append(Wp)
    return pl.pallas_call(
        functools.partial(_tc_layer_body, Wp is not None),
        grid=grid, in_specs=in_specs, out_specs=out_specs,
        out_shape=out_shape)(*args)


def _tc_final_body(a0, a1, t0, t1, deg, bl3, wr3l, wr3r, wreg, breg,
                   wd1, bd1, wd2, bd2, out):
    inv = 1.0 / jnp.maximum(deg[...], 1.0)
    m = (a0[...] + a1[...]) * inv
    h3 = m + bl3[...] + _dot_t(t0[...], wr3l[...]) + _dot_t(t1[...], wr3r[...])
    lat = _dot_t(h3, wreg[...]) + breg[...]
    d = jnp.maximum(_dot_t(lat, wd1[...]) + bd1[...], 0.0)
    out[...] = _dot_t(d, wd2[...]) + bd2[...]


def _tc_final(a0, a1, t0, t1, deg, bl3, Wr3, Wreg, breg, Wd1, bd1, Wd2, bd2):
    BN = 1024
    grid = (NPAD // BN,)
    in_specs = [pl.BlockSpec((BN, 128), lambda i: (i, 0)),
                pl.BlockSpec((BN, 128), lambda i: (i, 0)),
                pl.BlockSpec((BN, 128), lambda i: (i, 0)),
                pl.BlockSpec((BN, 128), lambda i: (i, 0)),
                pl.BlockSpec((BN, 1), lambda i: (i, 0)),
                pl.BlockSpec((1, 128), lambda i: (0, 0)),
                pl.BlockSpec((128, 128), lambda i: (0, 0)),
                pl.BlockSpec((128, 128), lambda i: (0, 0)),
                pl.BlockSpec((128, 128), lambda i: (0, 0)),
                pl.BlockSpec((1, 128), lambda i: (0, 0)),
                pl.BlockSpec((256, 128), lambda i: (0, 0)),
                pl.BlockSpec((1, 256), lambda i: (0, 0)),
                pl.BlockSpec((256, 256), lambda i: (0, 0)),
                pl.BlockSpec((1, 256), lambda i: (0, 0))]
    return pl.pallas_call(
        _tc_final_body, grid=grid, in_specs=in_specs,
        out_specs=pl.BlockSpec((BN, 256), lambda i: (i, 0)),
        out_shape=jax.ShapeDtypeStruct((NPAD, 256), jnp.float32),
    )(a0, a1, t0, t1, deg, bl3.reshape(1, 128),
      Wr3[:, :128], Wr3[:, 128:], Wreg, breg.reshape(1, 128),
      Wd1, bd1.reshape(1, 256), Wd2, bd2.reshape(1, 256))


def kernel(x, edge_index, Wl1, bl1, Wr1, Wl2, bl2, Wr2, Wl3, bl3, Wr3,
           Wreg, breg, Wd1, bd1, Wd2, bd2):
    ei = edge_index.astype(jnp.int32)
    pad_src = jnp.zeros((E_PAD - N_EDGES,), jnp.int32)
    pad_dst = jnp.full((E_PAD - N_EDGES,), NPAD - 1, jnp.int32)
    srcf = jnp.concatenate([ei[0], pad_src])
    dstf = jnp.concatenate([ei[1], pad_dst])
    src2d = srcf.reshape(TILES, NCH_A, K)
    dst2d = dstf.reshape(TILES, NCH_A, K)
    xp = jnp.pad(x, ((0, NPAD - N_NODES), (0, 0)))
    x0, x1 = xp[:, :128], xp[:, 128:]
    z128 = jnp.zeros((NPAD, 128), jnp.float32)
    z1 = jnp.zeros((NPAD,), jnp.float32)

    seg1_0, seg1_1, deg = _sc_agg(x0, x1, src2d, dst2d, z128, z1,
                                  with_deg=True)
    deg = deg.reshape(NPAD, 1)
    h1_0, h1_1 = _tc_layer(seg1_0, seg1_1, x0, x1, deg, Wl1, Wr1, bl1)
    seg2_0, seg2_1 = _sc_agg(h1_0, h1_1, src2d, dst2d, z128, z1,
                             with_deg=False)
    h2_0, h2_1, p = _tc_layer(seg2_0, seg2_1, h1_0, h1_1, deg,
                              Wl2, Wr2, bl2, Wp=Wl3)
    src3b = srcf.reshape(2 * TILES, NCH_B, K)
    dst3b = dstf.reshape(2 * TILES, NCH_B, K)
    seg3_0, seg3_1 = _sc_agg_edgesplit(p, src3b, dst3b, z128)
    out = _tc_final(seg3_0, seg3_1, h2_0, h2_1, deg, bl3, Wr3,
                    Wreg, breg, Wd1, bd1, Wd2, bd2)
    return out[:N_NODES]


# speedup vs baseline: 1.6415x; 1.6415x over previous
import functools

import jax
import jax.numpy as jnp
from jax import lax
from jax.experimental import pallas as pl
from jax.experimental.pallas import tpu as pltpu
from jax.experimental.pallas import tpu_sc as plsc

N_NODES = 10000
NPAD = 10240
N_EDGES = 160000
TILES = 16
K = 105
E_PAD = 161280
NCH_A = E_PAD // (TILES * K)
NPH_A = 4
NCH_B = E_PAD // (2 * TILES * K)
NPH_B = 2
CPP = 24
K_D = 128
E_PAD_D = 163840
NCH_D = E_PAD_D // (TILES * K_D)
NODE_SLICE = NPAD // TILES


def _pipe_agg(table, srcr, dstr, plane, nph, idx_v, rows_v, gsem, ssem,
              acc):
    def gather_desc(j, p):
        return pltpu.make_async_copy(table.at[idx_v.at[0, j]],
                                     rows_v.at[p], gsem)

    def scatter_start(j, p):
        pltpu.async_copy(rows_v.at[p], acc.at[idx_v.at[1, j]], ssem,
                         add=True)

    def scatter_wait(j, p):
        pltpu.make_async_copy(rows_v.at[p], acc.at[idx_v.at[1, j]],
                              ssem).wait()

    for ph in range(nph):
        pltpu.sync_copy(srcr.at[plane, pl.ds(ph * CPP, CPP)], idx_v.at[0])
        pltpu.sync_copy(dstr.at[plane, pl.ds(ph * CPP, CPP)], idx_v.at[1])
        gather_desc(0, 0).start()

        def body(j, carry):
            p = lax.bitwise_and(j, 1)
            gather_desc(j, p).wait()
            pl.when(j >= 1)(lambda: scatter_wait(j - 1, 1 - p))
            pl.when(j < CPP - 1)(lambda: gather_desc(j + 1, 1 - p).start())
            scatter_start(j, p)
            return carry

        lax.fori_loop(0, CPP, body, 0)
        scatter_wait(CPP - 1, (CPP - 1) & 1)


def _sc_agg(table0, table1, src3d, dst3d, zeros2d):
    mesh = plsc.VectorSubcoreMesh(core_axis_name="c", subcore_axis_name="s")
    out_type = [jax.ShapeDtypeStruct((NPAD, 128), jnp.float32),
                jax.ShapeDtypeStruct((NPAD, 128), jnp.float32)]
    scratch = [
        pltpu.VMEM((2, CPP, K), jnp.int32),
        pltpu.VMEM((2, K, 128), jnp.float32),
        pltpu.VMEM_SHARED((NPAD, 128), jnp.float32),
        pltpu.SemaphoreType.DMA,
        pltpu.SemaphoreType.DMA,
    ]

    def body(t0, t1, srcr, dstr, z2, o0, o1, idx_v, rows_v, acc, gsem, ssem):
        c = lax.axis_index("c")
        s = lax.axis_index("s")
        row0 = s * NODE_SLICE
        pltpu.sync_copy(z2.at[pl.ds(row0, NODE_SLICE)],
                        acc.at[pl.ds(row0, NODE_SLICE)])
        plsc.subcore_barrier()
        pl.when(c == 0)(lambda: _pipe_agg(t0, srcr, dstr, s, NPH_A,
                                          idx_v, rows_v, gsem, ssem, acc))
        pl.when(c == 1)(lambda: _pipe_agg(t1, srcr, dstr, s, NPH_A,
                                          idx_v, rows_v, gsem, ssem, acc))
        plsc.subcore_barrier()
        pl.when(c == 0)(lambda: pltpu.sync_copy(
            acc.at[pl.ds(row0, NODE_SLICE)], o0.at[pl.ds(row0, NODE_SLICE)]))
        pl.when(c == 1)(lambda: pltpu.sync_copy(
            acc.at[pl.ds(row0, NODE_SLICE)], o1.at[pl.ds(row0, NODE_SLICE)]))

    run_kernel = pl.kernel(body, out_type=out_type, mesh=mesh,
                           scratch_types=scratch)
    return run_kernel(table0, table1, src3d, dst3d, zeros2d)


def _sc_agg_edgesplit(table, src3d, dst3d, zeros2d):
    mesh = plsc.VectorSubcoreMesh(core_axis_name="c", subcore_axis_name="s")
    out_type = [jax.ShapeDtypeStruct((NPAD, 128), jnp.float32),
                jax.ShapeDtypeStruct((NPAD, 128), jnp.float32)]
    scratch = [
        pltpu.VMEM((2, CPP, K), jnp.int32),
        pltpu.VMEM((2, K, 128), jnp.float32),
        pltpu.VMEM_SHARED((NPAD, 128), jnp.float32),
        pltpu.SemaphoreType.DMA,
        pltpu.SemaphoreType.DMA,
    ]

    def body(t, srcr, dstr, z2, o0, o1, idx_v, rows_v, acc, gsem, ssem):
        c = lax.axis_index("c")
        s = lax.axis_index("s")
        q = c * TILES + s
        row0 = s * NODE_SLICE
        pltpu.sync_copy(z2.at[pl.ds(row0, NODE_SLICE)],
                        acc.at[pl.ds(row0, NODE_SLICE)])
        plsc.subcore_barrier()
        _pipe_agg(t, srcr, dstr, q, NPH_B, idx_v, rows_v, gsem, ssem, acc)
        plsc.subcore_barrier()
        pl.when(c == 0)(lambda: pltpu.sync_copy(
            acc.at[pl.ds(row0, NODE_SLICE)], o0.at[pl.ds(row0, NODE_SLICE)]))
        pl.when(c == 1)(lambda: pltpu.sync_copy(
            acc.at[pl.ds(row0, NODE_SLICE)], o1.at[pl.ds(row0, NODE_SLICE)]))

    run_kernel = pl.kernel(body, out_type=out_type, mesh=mesh,
                           scratch_types=scratch)
    return run_kernel(table, src3d, dst3d, zeros2d)


def _sc_deg(dst3d, zeros1d):
    mesh = plsc.VectorSubcoreMesh(core_axis_name="c", subcore_axis_name="s")
    out_type = jax.ShapeDtypeStruct((NPAD,), jnp.float32)
    scratch = [
        pltpu.VMEM((NCH_D, K_D), jnp.int32),
        pltpu.VMEM((K_D,), jnp.float32),
        pltpu.VMEM_SHARED((NPAD,), jnp.float32),
    ]

    def body(dstr, z1, degout, dst_v, ones_v, degacc):
        c = lax.axis_index("c")
        s = lax.axis_index("s")
        row0 = s * NODE_SLICE
        pltpu.sync_copy(z1.at[pl.ds(row0, NODE_SLICE)],
                        degacc.at[pl.ds(row0, NODE_SLICE)])
        pltpu.sync_copy(dstr.at[s], dst_v)
        for j in range(K_D // 16):
            ones_v[pl.ds(j * 16, 16)] = jnp.full((16,), 1.0, jnp.float32)
        plsc.subcore_barrier()

        def step(j, carry):
            pltpu.sync_copy(ones_v, degacc.at[dst_v.at[j]], add=True)
            return carry
        lax.fori_loop(0, NCH_D, step, 0)
        plsc.subcore_barrier()

        @pl.when(c == 0)
        def _():
            pltpu.sync_copy(degacc.at[pl.ds(row0, NODE_SLICE)],
                            degout.at[pl.ds(row0, NODE_SLICE)])

    run_kernel = pl.kernel(body, out_type=out_type, mesh=mesh,
                           scratch_types=scratch)
    return run_kernel(dst3d, zeros1d)


def _dot_t(a, w):
    return lax.dot_general(a, w, (((1,), (1,)), ((), ())),
                           preferred_element_type=jnp.float32)


def _tc_layer_body(with_p, *refs):
    if with_p:
        (a0, a1, t0, t1, deg, wll, wlr, wrl, wrr, b, wp,
         o0, o1, po) = refs
    else:
        a0, a1, t0, t1, deg, wll, wlr, wrl, wrr, b, o0, o1 = refs
    inv = 1.0 / jnp.maximum(deg[...], 1.0)
    h = (_dot_t(a0[...] * inv, wll[...]) + _dot_t(a1[...] * inv, wlr[...])
         + _dot_t(t0[...], wrl[...]) + _dot_t(t1[...], wrr[...]) + b[...])
    h = jnp.maximum(h, 0.0)
    o0[...] = h[:, :128]
    o1[...] = h[:, 128:]
    if with_p:
        po[...] = _dot_t(h, wp[...])


def _tc_layer(a0, a1, t0, t1, deg, Wl, Wr, b, Wp=None):
    BN = 1024
    grid = (NPAD // BN,)
    half = pl.BlockSpec((BN, 128), lambda i: (i, 0))
    wspec = pl.BlockSpec((256, 128), lambda i: (0, 0))
    in_specs = [half, half, half, half,
                pl.BlockSpec((BN, 1), lambda i: (i, 0)),
                wspec, wspec, wspec, wspec,
                pl.BlockSpec((1, 256), lambda i: (0, 0))]
    out_shape = [jax.ShapeDtypeStruct((NPAD, 128), jnp.float32),
                 jax.ShapeDtypeStruct((NPAD, 128), jnp.float32)]
    out_specs = [half, half]
    args = [a0, a1, t0, t1, deg,
            Wl[:, :128], Wl[:, 128:], Wr[:, :128], Wr[:, 128:],
            b.reshape(1, 256)]
    if Wp is not None:
        in_specs.append(pl.BlockSpec((128, 256), lambda i: (0, 0)))
        out_shape.append(jax.ShapeDtypeStruct((NPAD, 128), jnp.float32))
        out_specs.append(half)
        args.append(Wp)
    return pl.pallas_call(
        functools.partial(_tc_layer_body, Wp is not None),
        grid=grid, in_specs=in_specs, out_specs=out_specs,
        out_shape=out_shape)(*args)


def _tc_final_body(a0, a1, t0, t1, deg, bl3, wr3l, wr3r, wreg, breg,
                   wd1, bd1, wd2, bd2, out):
    inv = 1.0 / jnp.maximum(deg[...], 1.0)
    m = (a0[...] + a1[...]) * inv
    h3 = m + bl3[...] + _dot_t(t0[...], wr3l[...]) + _dot_t(t1[...], wr3r[...])
    lat = _dot_t(h3, wreg[...]) + breg[...]
    d = jnp.maximum(_dot_t(lat, wd1[...]) + bd1[...], 0.0)
    out[...] = _dot_t(d, wd2[...]) + bd2[...]


def _tc_final(a0, a1, t0, t1, deg, bl3, Wr3, Wreg, breg, Wd1, bd1, Wd2, bd2):
    BN = 1024
    grid = (NPAD // BN,)
    in_specs = [pl.BlockSpec((BN, 128), lambda i: (i, 0)),
                pl.BlockSpec((BN, 128), lambda i: (i, 0)),
                pl.BlockSpec((BN, 128), lambda i: (i, 0)),
                pl.BlockSpec((BN, 128), lambda i: (i, 0)),
                pl.BlockSpec((BN, 1), lambda i: (i, 0)),
                pl.BlockSpec((1, 128), lambda i: (0, 0)),
                pl.BlockSpec((128, 128), lambda i: (0, 0)),
                pl.BlockSpec((128, 128), lambda i: (0, 0)),
                pl.BlockSpec((128, 128), lambda i: (0, 0)),
                pl.BlockSpec((1, 128), lambda i: (0, 0)),
                pl.BlockSpec((256, 128), lambda i: (0, 0)),
                pl.BlockSpec((1, 256), lambda i: (0, 0)),
                pl.BlockSpec((256, 256), lambda i: (0, 0)),
                pl.BlockSpec((1, 256), lambda i: (0, 0))]
    return pl.pallas_call(
        _tc_final_body, grid=grid, in_specs=in_specs,
        out_specs=pl.BlockSpec((BN, 256), lambda i: (i, 0)),
        out_shape=jax.ShapeDtypeStruct((NPAD, 256), jnp.float32),
    )(a0, a1, t0, t1, deg, bl3.reshape(1, 128),
      Wr3[:, :128], Wr3[:, 128:], Wreg, breg.reshape(1, 128),
      Wd1, bd1.reshape(1, 256), Wd2, bd2.reshape(1, 256))


def kernel(x, edge_index, Wl1, bl1, Wr1, Wl2, bl2, Wr2, Wl3, bl3, Wr3,
           Wreg, breg, Wd1, bd1, Wd2, bd2):
    ei = edge_index.astype(jnp.int32)
    def pad_edges(n):
        s = jnp.concatenate([ei[0], jnp.zeros((n - N_EDGES,), jnp.int32)])
        d = jnp.concatenate([ei[1], jnp.full((n - N_EDGES,), NPAD - 1,
                                             jnp.int32)])
        return s, d
    srca, dsta = pad_edges(E_PAD)
    src2d = srca.reshape(TILES, NCH_A, K)
    dst2d = dsta.reshape(TILES, NCH_A, K)
    src3b = srca.reshape(2 * TILES, NCH_B, K)
    dst3b = dsta.reshape(2 * TILES, NCH_B, K)
    _, dstd = pad_edges(E_PAD_D)
    dst3d = dstd.reshape(TILES, NCH_D, K_D)
    xp = jnp.pad(x, ((0, NPAD - N_NODES), (0, 0)))
    x0, x1 = xp[:, :128], xp[:, 128:]
    z128 = jnp.zeros((NPAD, 128), jnp.float32)
    z1 = jnp.zeros((NPAD,), jnp.float32)

    deg = _sc_deg(dst3d, z1).reshape(NPAD, 1)
    seg1_0, seg1_1 = _sc_agg(x0, x1, src2d, dst2d, z128)
    h1_0, h1_1 = _tc_layer(seg1_0, seg1_1, x0, x1, deg, Wl1, Wr1, bl1)
    seg2_0, seg2_1 = _sc_agg(h1_0, h1_1, src2d, dst2d, z128)
    h2_0, h2_1, p = _tc_layer(seg2_0, seg2_1, h1_0, h1_1, deg,
                              Wl2, Wr2, bl2, Wp=Wl3)
    seg3_0, seg3_1 = _sc_agg_edgesplit(p, src3b, dst3b, z128)
    out = _tc_final(seg3_0, seg3_1, h2_0, h2_1, deg, bl3, Wr3,
                    Wreg, breg, Wd1, bd1, Wd2, bd2)
    return out[:N_NODES]
